# baseline (speedup 1.0000x reference)
import functools

import jax
import jax.numpy as jnp
from jax import lax
from jax.experimental import pallas as pl
from jax.experimental.pallas import tpu as pltpu
from jax.experimental.pallas import tpu_sc as plsc

VOCAB = 100000
EMBED = 64
BATCH = 1024
CTX = 50

_NC = 2
_NS = 16
_NW = _NC * _NS
_BPW = BATCH // _NW
_RPW = _BPW * CTX
_GCH = 80
_NCHUNK = _RPW // _GCH


def _rsqrt_newton(x):
    i = plsc.bitcast(x, jnp.int32)
    i = jnp.int32(0x5F3759DF) - lax.shift_right_logical(i, 1)
    y = plsc.bitcast(i, jnp.float32)
    for _ in range(3):
        y = y * (1.5 - 0.5 * x * y * y)
    return y


def _make_pool_kernel():
    mesh = plsc.VectorSubcoreMesh(core_axis_name="c", subcore_axis_name="s")

    @functools.partial(
        pl.kernel,
        out_type=jax.ShapeDtypeStruct((BATCH, EMBED), jnp.float32),
        mesh=mesh,
        compiler_params=pltpu.CompilerParams(
            needs_layout_passes=False, use_tc_tiling_on_sc=False
        ),
        scratch_types=[
            pltpu.VMEM((_RPW,), jnp.int32),
            pltpu.VMEM((_RPW, EMBED), jnp.float32),
            pltpu.VMEM((_BPW, EMBED), jnp.float32),
            pltpu.SemaphoreType.DMA,
        ],
    )
    def pool(table_hbm, idx_hbm, out_hbm, idx_v, rows_v, pool_v, sem):
        wid = lax.axis_index("s") * _NC + lax.axis_index("c")
        pltpu.sync_copy(idx_hbm.at[pl.ds(wid * _RPW, _RPW)], idx_v)
        copies = []
        for k in range(_NCHUNK):
            cp = pltpu.make_async_copy(
                table_hbm.at[idx_v.at[pl.ds(k * _GCH, _GCH)]],
                rows_v.at[pl.ds(k * _GCH, _GCH)],
                sem,
            )
            cp.start()
            copies.append(cp)
        for cp in copies:
            cp.wait()

        lanes = lax.iota(jnp.int32, 16)

        def norm_body(g, _):
            r0 = pl.multiple_of(g * 16, 16)
            rowidx = r0 + lanes

            def col_body(k, sqacc):
                colidx = jnp.full((16,), k, dtype=jnp.int32)
                v = plsc.load_gather(rows_v, [rowidx, colidx])
                return sqacc + v * v

            n2 = lax.fori_loop(0, EMBED, col_body, jnp.zeros((16,), jnp.float32))
            scale = jnp.where(n2 > 1.0, _rsqrt_newton(n2), 1.0)
            for i in range(16):
                s = scale[i]
                for j in range(EMBED // 16):
                    sl = (r0 + i, pl.ds(j * 16, 16))
                    rows_v[sl] = rows_v[sl] * s
            return 0

        lax.fori_loop(0, _RPW // 16, norm_body, 0)

        def row_body(b, _):
            def ctx_body(c, acc):
                r = b * CTX + c
                vs = [rows_v[r, pl.ds(j * 16, 16)] for j in range(EMBED // 16)]
                return tuple(a + v for a, v in zip(acc, vs))

            zero = jnp.zeros((16,), jnp.float32)
            acc = lax.fori_loop(0, CTX, ctx_body, (zero,) * (EMBED // 16))
            inv = jnp.float32(1.0 / CTX)
            for j in range(EMBED // 16):
                pool_v[b, pl.ds(j * 16, 16)] = acc[j] * inv
            return 0

        lax.fori_loop(0, _BPW, row_body, 0)
        pltpu.sync_copy(pool_v, out_hbm.at[pl.ds(wid * _BPW, _BPW)])

    return pool


_pool_kernel = _make_pool_kernel()

_VB = 2048
_NFULL = VOCAB // _VB


def _mm_body(p_ref, w_ref, b_ref, o_ref):
    o_ref[...] = (
        lax.dot_general(
            p_ref[...],
            w_ref[...],
            (((1,), (1,)), ((), ())),
            preferred_element_type=jnp.float32,
        )
        + b_ref[...]
    )


def _mm_tail_body(p_ref, w_ref, b_ref, buf_ref, o_ref):
    del buf_ref
    _mm_body(p_ref, w_ref, b_ref, o_ref)


def _project(pooled, lin_w, lin_b2d):
    main = pl.pallas_call(
        _mm_body,
        grid=(_NFULL,),
        in_specs=[
            pl.BlockSpec((BATCH, EMBED), lambda j: (0, 0)),
            pl.BlockSpec((_VB, EMBED), lambda j: (j, 0)),
            pl.BlockSpec((1, _VB), lambda j: (0, j)),
        ],
        out_specs=pl.BlockSpec((BATCH, _VB), lambda j: (0, j)),
        out_shape=jax.ShapeDtypeStruct((BATCH, VOCAB), jnp.float32),
        compiler_params=pltpu.CompilerParams(
            dimension_semantics=("parallel",),
        ),
    )(pooled, lin_w, lin_b2d)
    return pl.pallas_call(
        _mm_tail_body,
        grid=(1,),
        in_specs=[
            pl.BlockSpec((BATCH, EMBED), lambda j: (0, 0)),
            pl.BlockSpec((_VB, EMBED), lambda j: (_NFULL, 0)),
            pl.BlockSpec((1, _VB), lambda j: (0, _NFULL)),
            pl.BlockSpec(memory_space=pl.ANY),
        ],
        out_specs=pl.BlockSpec((BATCH, _VB), lambda j: (0, _NFULL)),
        out_shape=jax.ShapeDtypeStruct((BATCH, VOCAB), jnp.float32),
        input_output_aliases={3: 0},
        compiler_params=pltpu.CompilerParams(
            dimension_semantics=("arbitrary",),
        ),
    )(pooled, lin_w, lin_b2d, main)


def kernel(inputs_, emb_table, lin_w, lin_b):
    idx = inputs_.astype(jnp.int32).reshape(-1)
    pooled = _pool_kernel(emb_table, idx)
    return _project(pooled, lin_w, lin_b.reshape(1, VOCAB))


# speedup vs baseline: 1.0010x; 1.0010x over previous
import functools

import jax
import jax.numpy as jnp
from jax import lax
from jax.experimental import pallas as pl
from jax.experimental.pallas import tpu as pltpu
from jax.experimental.pallas import tpu_sc as plsc

VOCAB = 100000
EMBED = 64
BATCH = 1024
CTX = 50

_NC = 2
_NS = 16
_NW = _NC * _NS
_BPW = BATCH // _NW
_RPW = _BPW * CTX
_GCH = 80
_NCHUNK = _RPW // _GCH


def _rsqrt_newton(x):
    i = plsc.bitcast(x, jnp.int32)
    i = jnp.int32(0x5F3759DF) - lax.shift_right_logical(i, 1)
    y = plsc.bitcast(i, jnp.float32)
    for _ in range(3):
        y = y * (1.5 - 0.5 * x * y * y)
    return y


def _make_pool_kernel():
    mesh = plsc.VectorSubcoreMesh(core_axis_name="c", subcore_axis_name="s")

    @functools.partial(
        pl.kernel,
        out_type=jax.ShapeDtypeStruct((BATCH, EMBED), jnp.float32),
        mesh=mesh,
        compiler_params=pltpu.CompilerParams(
            needs_layout_passes=False, use_tc_tiling_on_sc=False
        ),
        scratch_types=[
            pltpu.VMEM((_RPW,), jnp.int32),
            pltpu.VMEM((_RPW, EMBED), jnp.float32),
            pltpu.VMEM((_BPW, EMBED), jnp.float32),
            pltpu.SemaphoreType.DMA,
        ],
    )
    def pool(table_hbm, idx_hbm, out_hbm, idx_v, rows_v, pool_v, sem):
        wid = lax.axis_index("s") * _NC + lax.axis_index("c")
        pltpu.sync_copy(idx_hbm.at[pl.ds(wid * _RPW, _RPW)], idx_v)
        copies = []
        for k in range(_NCHUNK):
            cp = pltpu.make_async_copy(
                table_hbm.at[idx_v.at[pl.ds(k * _GCH, _GCH)]],
                rows_v.at[pl.ds(k * _GCH, _GCH)],
                sem,
            )
            cp.start()
            copies.append(cp)
        for cp in copies:
            cp.wait()

        lanes = lax.iota(jnp.int32, 16)

        def norm_body(g, _):
            r0 = pl.multiple_of(g * 16, 16)
            rowidx = r0 + lanes

            def col_body(k, sqacc):
                colidx = jnp.full((16,), k, dtype=jnp.int32)
                v = plsc.load_gather(rows_v, [rowidx, colidx])
                return sqacc + v * v

            n2 = lax.fori_loop(0, EMBED, col_body, jnp.zeros((16,), jnp.float32))
            scale = jnp.where(n2 > 1.0, _rsqrt_newton(n2), 1.0)
            for i in range(16):
                s = scale[i]
                for j in range(EMBED // 16):
                    sl = (r0 + i, pl.ds(j * 16, 16))
                    rows_v[sl] = rows_v[sl] * s
            return 0

        lax.fori_loop(0, _RPW // 16, norm_body, 0)

        def row_body(b, _):
            def ctx_body(c, acc):
                r = b * CTX + c
                vs = [rows_v[r, pl.ds(j * 16, 16)] for j in range(EMBED // 16)]
                return tuple(a + v for a, v in zip(acc, vs))

            zero = jnp.zeros((16,), jnp.float32)
            acc = lax.fori_loop(0, CTX, ctx_body, (zero,) * (EMBED // 16))
            inv = jnp.float32(1.0 / CTX)
            for j in range(EMBED // 16):
                pool_v[b, pl.ds(j * 16, 16)] = acc[j] * inv
            return 0

        lax.fori_loop(0, _BPW, row_body, 0)
        pltpu.sync_copy(pool_v, out_hbm.at[pl.ds(wid * _BPW, _BPW)])

    return pool


_pool_kernel = _make_pool_kernel()

_VB = 2048
_NFULL = VOCAB // _VB


def _mm_body(p_ref, w_ref, b_ref, o_ref):
    o_ref[...] = (
        lax.dot_general(
            p_ref[...],
            w_ref[...],
            (((1,), (1,)), ((), ())),
            preferred_element_type=jnp.float32,
        )
        + b_ref[...]
    )


def _mm_manual(p_ref, w_ref, b_ref, out_hbm, buf0, buf1, sems):
    j = pl.program_id(0)
    x = (
        lax.dot_general(
            p_ref[...],
            w_ref[...],
            (((1,), (1,)), ((), ())),
            preferred_element_type=jnp.float32,
        )
        + b_ref[...]
    )

    def dma(slot, buf, jj):
        return pltpu.make_async_copy(
            buf,
            out_hbm.at[:, pl.ds(jj * _VB, _VB)],
            sems.at[slot],
        )

    for slot, buf in ((0, buf0), (1, buf1)):
        is_mine = lax.rem(j, 2) == slot

        @pl.when(jnp.logical_and(is_mine, j >= 2))
        def _():
            dma(slot, buf, j - 2).wait()

        @pl.when(is_mine)
        def _():
            buf[...] = x
            dma(slot, buf, j).start()

    @pl.when(j == _NFULL - 1)
    def _():
        dma(0, buf0, 0).wait()
        dma(1, buf1, 0).wait()


def _mm_tail_body(p_ref, w_ref, b_ref, buf_ref, o_ref):
    del buf_ref
    _mm_body(p_ref, w_ref, b_ref, o_ref)


def _project(pooled, lin_w, lin_b2d):
    main = pl.pallas_call(
        _mm_manual,
        grid=(_NFULL,),
        in_specs=[
            pl.BlockSpec((BATCH, EMBED), lambda j: (0, 0)),
            pl.BlockSpec((_VB, EMBED), lambda j: (j, 0)),
            pl.BlockSpec((1, _VB), lambda j: (0, j)),
        ],
        out_specs=pl.BlockSpec(memory_space=pl.ANY),
        out_shape=jax.ShapeDtypeStruct((BATCH, VOCAB), jnp.float32),
        scratch_shapes=[
            pltpu.VMEM((BATCH, _VB), jnp.float32),
            pltpu.VMEM((BATCH, _VB), jnp.float32),
            pltpu.SemaphoreType.DMA((2,)),
        ],
        compiler_params=pltpu.CompilerParams(
            dimension_semantics=("arbitrary",),
        ),
    )(pooled, lin_w, lin_b2d)
    return pl.pallas_call(
        _mm_tail_body,
        grid=(1,),
        in_specs=[
            pl.BlockSpec((BATCH, EMBED), lambda j: (0, 0)),
            pl.BlockSpec((_VB, EMBED), lambda j: (_NFULL, 0)),
            pl.BlockSpec((1, _VB), lambda j: (0, _NFULL)),
            pl.BlockSpec(memory_space=pl.ANY),
        ],
        out_specs=pl.BlockSpec((BATCH, _VB), lambda j: (0, _NFULL)),
        out_shape=jax.ShapeDtypeStruct((BATCH, VOCAB), jnp.float32),
        input_output_aliases={3: 0},
        compiler_params=pltpu.CompilerParams(
            dimension_semantics=("arbitrary",),
        ),
    )(pooled, lin_w, lin_b2d, main)


def kernel(inputs_, emb_table, lin_w, lin_b):
    idx = inputs_.astype(jnp.int32).reshape(-1)
    pooled = _pool_kernel(emb_table, idx)
    return _project(pooled, lin_w, lin_b.reshape(1, VOCAB))


# speedup vs baseline: 2.2177x; 2.2154x over previous
import functools

import jax
import jax.numpy as jnp
from jax import lax
from jax.experimental import pallas as pl
from jax.experimental.pallas import tpu as pltpu
from jax.experimental.pallas import tpu_sc as plsc

VOCAB = 100000
EMBED = 64
BATCH = 1024
CTX = 50

_NC = 2
_NS = 16
_NW = _NC * _NS
_BPW = BATCH // _NW
_RPW = _BPW * CTX
_GCH = 80
_NCHUNK = _RPW // _GCH


def _rsqrt_newton(x):
    i = plsc.bitcast(x, jnp.int32)
    i = jnp.int32(0x5F3759DF) - lax.shift_right_logical(i, 1)
    y = plsc.bitcast(i, jnp.float32)
    for _ in range(3):
        y = y * (1.5 - 0.5 * x * y * y)
    return y


def _make_pool_kernel():
    mesh = plsc.VectorSubcoreMesh(core_axis_name="c", subcore_axis_name="s")

    @functools.partial(
        pl.kernel,
        out_type=jax.ShapeDtypeStruct((BATCH, EMBED), jnp.float32),
        mesh=mesh,
        compiler_params=pltpu.CompilerParams(
            needs_layout_passes=False, use_tc_tiling_on_sc=False
        ),
        scratch_types=[
            pltpu.VMEM((_RPW,), jnp.int32),
            pltpu.VMEM((_RPW, EMBED), jnp.float32),
            pltpu.VMEM((_BPW, EMBED), jnp.float32),
            pltpu.SemaphoreType.DMA,
        ],
    )
    def pool(table_hbm, idx_hbm, out_hbm, idx_v, rows_v, pool_v, sem):
        wid = lax.axis_index("s") * _NC + lax.axis_index("c")
        pltpu.sync_copy(idx_hbm.at[pl.ds(wid * _RPW, _RPW)], idx_v)
        copies = []
        for k in range(_NCHUNK):
            cp = pltpu.make_async_copy(
                table_hbm.at[idx_v.at[pl.ds(k * _GCH, _GCH)]],
                rows_v.at[pl.ds(k * _GCH, _GCH)],
                sem,
            )
            cp.start()
            copies.append(cp)
        for cp in copies:
            cp.wait()

        lanes = lax.iota(jnp.int32, 16)

        def norm_body(g, _):
            r0 = pl.multiple_of(g * 16, 16)
            rowidx = r0 + lanes

            def col_body(k, sqacc):
                colidx = jnp.full((16,), k, dtype=jnp.int32)
                v = plsc.load_gather(rows_v, [rowidx, colidx])
                return sqacc + v * v

            n2 = lax.fori_loop(0, EMBED, col_body, jnp.zeros((16,), jnp.float32))
            scale = jnp.where(n2 > 1.0, _rsqrt_newton(n2), 1.0)
            for i in range(16):
                s = scale[i]
                for j in range(EMBED // 16):
                    sl = (r0 + i, pl.ds(j * 16, 16))
                    rows_v[sl] = rows_v[sl] * s
            return 0

        lax.fori_loop(0, _RPW // 16, norm_body, 0)

        def row_body(b, _):
            def ctx_body(c, acc):
                r = b * CTX + c
                vs = [rows_v[r, pl.ds(j * 16, 16)] for j in range(EMBED // 16)]
                return tuple(a + v for a, v in zip(acc, vs))

            zero = jnp.zeros((16,), jnp.float32)
            acc = lax.fori_loop(0, CTX, ctx_body, (zero,) * (EMBED // 16))
            inv = jnp.float32(1.0 / CTX)
            for j in range(EMBED // 16):
                pool_v[b, pl.ds(j * 16, 16)] = acc[j] * inv
            return 0

        lax.fori_loop(0, _BPW, row_body, 0)
        pltpu.sync_copy(pool_v, out_hbm.at[pl.ds(wid * _BPW, _BPW)])

    return pool


_pool_kernel = _make_pool_kernel()

_VB = 2048
_NFULL = VOCAB // _VB


def _mm_body(p_ref, w_ref, b_ref, o_ref):
    o_ref[...] = (
        lax.dot_general(
            p_ref[...],
            w_ref[...],
            (((1,), (1,)), ((), ())),
            preferred_element_type=jnp.float32,
        )
        + b_ref[...]
    )


_NBT = _NFULL + 1
_TAILR = VOCAB - _NFULL * _VB


def _mmt_body(p_ref, wt_ref, b_ref, out_hbm, buf0, buf1, sems):
    j = pl.program_id(0)
    xt = lax.dot_general(
        wt_ref[...],
        p_ref[...],
        (((0,), (1,)), ((), ())),
        preferred_element_type=jnp.float32,
    ) + jnp.transpose(b_ref[...], (1, 0))

    def dma(slot, buf, jj, rows):
        return pltpu.make_async_copy(
            buf.at[pl.ds(0, rows)],
            out_hbm.at[pl.ds(jj * _VB, rows)],
            sems.at[slot],
        )

    for slot, buf in ((0, buf0), (1, buf1)):
        is_mine = lax.rem(j, 2) == slot

        @pl.when(jnp.logical_and(is_mine, j >= 2))
        def _():
            dma(slot, buf, j - 2, _VB).wait()

        @pl.when(is_mine)
        def _():
            buf[...] = xt

        @pl.when(jnp.logical_and(is_mine, j < _NBT - 1))
        def _():
            dma(slot, buf, j, _VB).start()

        @pl.when(jnp.logical_and(is_mine, j == _NBT - 1))
        def _():
            dma(slot, buf, j, _TAILR).start()

    @pl.when(j == _NBT - 1)
    def _():
        dma(1, buf1, 0, _VB).wait()
        dma(0, buf0, 0, _TAILR).wait()


def _project(pooled, lin_wt, lin_b2d):
    out_t = pl.pallas_call(
        _mmt_body,
        grid=(_NBT,),
        in_specs=[
            pl.BlockSpec((BATCH, EMBED), lambda j: (0, 0)),
            pl.BlockSpec((EMBED, _VB), lambda j: (0, j)),
            pl.BlockSpec((1, _VB), lambda j: (0, j)),
        ],
        out_specs=pl.BlockSpec(memory_space=pl.ANY),
        out_shape=jax.ShapeDtypeStruct((VOCAB, BATCH), jnp.float32),
        scratch_shapes=[
            pltpu.VMEM((_VB, BATCH), jnp.float32),
            pltpu.VMEM((_VB, BATCH), jnp.float32),
            pltpu.SemaphoreType.DMA((2,)),
        ],
        compiler_params=pltpu.CompilerParams(
            dimension_semantics=("arbitrary",),
        ),
    )(pooled, lin_wt, lin_b2d)
    return out_t.T


def kernel(inputs_, emb_table, lin_w, lin_b):
    idx = inputs_.astype(jnp.int32).reshape(-1)
    pooled = _pool_kernel(emb_table, idx)
    return _project(pooled, lin_w.T, lin_b.reshape(1, VOCAB))


# speedup vs baseline: 2.8249x; 1.2738x over previous
import functools

import jax
import jax.numpy as jnp
from jax import lax
from jax.experimental import pallas as pl
from jax.experimental.pallas import tpu as pltpu
from jax.experimental.pallas import tpu_sc as plsc

VOCAB = 100000
EMBED = 64
BATCH = 1024
CTX = 50

_NC = 2
_NS = 16
_NW = _NC * _NS
_BPW = BATCH // _NW
_RPW = _BPW * CTX
_GCH = 80
_NCHUNK = _RPW // _GCH


def _rsqrt_newton(x):
    i = plsc.bitcast(x, jnp.int32)
    i = jnp.int32(0x5F3759DF) - lax.shift_right_logical(i, 1)
    y = plsc.bitcast(i, jnp.float32)
    for _ in range(3):
        y = y * (1.5 - 0.5 * x * y * y)
    return y


def _make_pool_kernel():
    mesh = plsc.VectorSubcoreMesh(core_axis_name="c", subcore_axis_name="s")

    @functools.partial(
        pl.kernel,
        out_type=jax.ShapeDtypeStruct((BATCH, EMBED), jnp.float32),
        mesh=mesh,
        compiler_params=pltpu.CompilerParams(
            needs_layout_passes=False, use_tc_tiling_on_sc=False
        ),
        scratch_types=[
            pltpu.VMEM((_RPW,), jnp.int32),
            pltpu.VMEM((_RPW, EMBED), jnp.float32),
            pltpu.VMEM((_BPW, EMBED), jnp.float32),
            pltpu.SemaphoreType.DMA,
        ],
    )
    def pool(table_hbm, idx_hbm, out_hbm, idx_v, rows_v, pool_v, sem):
        wid = lax.axis_index("s") * _NC + lax.axis_index("c")
        pltpu.sync_copy(idx_hbm.at[pl.ds(wid * _RPW, _RPW)], idx_v)
        copies = []
        for k in range(_NCHUNK):
            cp = pltpu.make_async_copy(
                table_hbm.at[idx_v.at[pl.ds(k * _GCH, _GCH)]],
                rows_v.at[pl.ds(k * _GCH, _GCH)],
                sem,
            )
            cp.start()
            copies.append(cp)
        for cp in copies:
            cp.wait()

        def row_body(b, _):
            def ctx_body(c, acc):
                r = b * CTX + c
                vs = [rows_v[r, pl.ds(j * 16, 16)] for j in range(EMBED // 16)]
                sq = vs[0] * vs[0]
                for v in vs[1:]:
                    sq = sq + v * v
                n2 = jnp.full((16,), jnp.sum(sq), dtype=jnp.float32)
                scale = jnp.where(n2 > 1.0, _rsqrt_newton(n2), 1.0)
                return tuple(a + scale * v for a, v in zip(acc, vs))

            zero = jnp.zeros((16,), jnp.float32)
            acc = lax.fori_loop(0, CTX, ctx_body, (zero,) * (EMBED // 16),
                                unroll=5)
            inv = jnp.float32(1.0 / CTX)
            for j in range(EMBED // 16):
                pool_v[b, pl.ds(j * 16, 16)] = acc[j] * inv
            return 0

        lax.fori_loop(0, _BPW, row_body, 0)
        pltpu.sync_copy(pool_v, out_hbm.at[pl.ds(wid * _BPW, _BPW)])

    return pool


_pool_kernel = _make_pool_kernel()

_VB = 2048
_NFULL = VOCAB // _VB


def _mm_body(p_ref, w_ref, b_ref, o_ref):
    o_ref[...] = (
        lax.dot_general(
            p_ref[...],
            w_ref[...],
            (((1,), (1,)), ((), ())),
            preferred_element_type=jnp.float32,
        )
        + b_ref[...]
    )


_NBT = _NFULL + 1
_TAILR = VOCAB - _NFULL * _VB


def _mmt_body(p_ref, wt_ref, b_ref, out_hbm, buf0, buf1, sems):
    j = pl.program_id(0)
    xt = lax.dot_general(
        wt_ref[...],
        p_ref[...],
        (((0,), (1,)), ((), ())),
        preferred_element_type=jnp.float32,
    ) + jnp.transpose(b_ref[...], (1, 0))

    def dma(slot, buf, jj, rows):
        return pltpu.make_async_copy(
            buf.at[pl.ds(0, rows)],
            out_hbm.at[pl.ds(jj * _VB, rows)],
            sems.at[slot],
        )

    for slot, buf in ((0, buf0), (1, buf1)):
        is_mine = lax.rem(j, 2) == slot

        @pl.when(jnp.logical_and(is_mine, j >= 2))
        def _():
            dma(slot, buf, j - 2, _VB).wait()

        @pl.when(is_mine)
        def _():
            buf[...] = xt

        @pl.when(jnp.logical_and(is_mine, j < _NBT - 1))
        def _():
            dma(slot, buf, j, _VB).start()

        @pl.when(jnp.logical_and(is_mine, j == _NBT - 1))
        def _():
            dma(slot, buf, j, _TAILR).start()

    @pl.when(j == _NBT - 1)
    def _():
        dma(1, buf1, 0, _VB).wait()
        dma(0, buf0, 0, _TAILR).wait()


def _project(pooled, lin_wt, lin_b2d):
    out_t = pl.pallas_call(
        _mmt_body,
        grid=(_NBT,),
        in_specs=[
            pl.BlockSpec((BATCH, EMBED), lambda j: (0, 0)),
            pl.BlockSpec((EMBED, _VB), lambda j: (0, j)),
            pl.BlockSpec((1, _VB), lambda j: (0, j)),
        ],
        out_specs=pl.BlockSpec(memory_space=pl.ANY),
        out_shape=jax.ShapeDtypeStruct((VOCAB, BATCH), jnp.float32),
        scratch_shapes=[
            pltpu.VMEM((_VB, BATCH), jnp.float32),
            pltpu.VMEM((_VB, BATCH), jnp.float32),
            pltpu.SemaphoreType.DMA((2,)),
        ],
        compiler_params=pltpu.CompilerParams(
            dimension_semantics=("arbitrary",),
        ),
    )(pooled, lin_wt, lin_b2d)
    return out_t.T


def kernel(inputs_, emb_table, lin_w, lin_b):
    idx = inputs_.astype(jnp.int32).reshape(-1)
    pooled = _pool_kernel(emb_table, idx)
    return _project(pooled, lin_w.T, lin_b.reshape(1, VOCAB))
